# baseline (speedup 1.0000x reference)
import functools

import jax
import jax.numpy as jnp
from jax import lax
from jax.experimental import pallas as pl
from jax.experimental.pallas import tpu as pltpu
from jax.experimental.pallas import tpu_sc as plsc

N = 10000
E = 320000
C = 128
HID = 256
NT = 10
NTYPES = 32
NF = 16
NG = 64
EPS = 1e-5

NCORES = 2
NSUB = 16
NWORK = NCORES * NSUB
CHUNK = 128
NBUF = 2
EPW = 10240
EPAD = EPW * NWORK
NCHUNK = EPW // CHUNK
NGRP = NCHUNK // NBUF
NITER = NGRP // 2
ACC_N = 10240
RPT = ACC_N // NSUB
DUMMY = N


def _spmm_body(zs_hbm, srcp, dstp, zeros_hbm, out_hbm,
               isrc, idst, rows, acc_sh, isem, gsem, ssem):
    c = lax.axis_index("c")
    s = lax.axis_index("s")
    w = c * NSUB + s
    for p in range(2):
        pltpu.async_copy(srcp.at[w, pl.ds(p * NBUF, NBUF)], isrc.at[p],
                         isem.at[p, 0])
        pltpu.async_copy(dstp.at[w, pl.ds(p * NBUF, NBUF)], idst.at[p],
                         isem.at[p, 1])
    pltpu.sync_copy(zeros_hbm, acc_sh.at[pl.ds(s * RPT, RPT)])
    plsc.subcore_barrier()

    def it(k, carry):
        for p in range(2):
            g = 2 * k + p
            pltpu.make_async_copy(srcp.at[w, pl.ds(g * NBUF, NBUF)],
                                  isrc.at[p], isem.at[p, 0]).wait()
            pltpu.make_async_copy(dstp.at[w, pl.ds(g * NBUF, NBUF)],
                                  idst.at[p], isem.at[p, 1]).wait()
            for b in range(NBUF):
                pltpu.async_copy(zs_hbm.at[isrc.at[p, b]], rows.at[b],
                                 gsem.at[b])
            for b in range(NBUF):
                pltpu.make_async_copy(zs_hbm.at[isrc.at[p, b]], rows.at[b],
                                      gsem.at[b]).wait()
                pltpu.async_copy(rows.at[b], acc_sh.at[idst.at[p, b]],
                                 ssem.at[b], add=True)
            for b in range(NBUF):
                pltpu.make_async_copy(rows.at[b], acc_sh.at[idst.at[p, b]],
                                      ssem.at[b]).wait()
            @pl.when(g + 2 < NGRP)
            def _():
                pltpu.async_copy(srcp.at[w, pl.ds((g + 2) * NBUF, NBUF)],
                                 isrc.at[p], isem.at[p, 0])
                pltpu.async_copy(dstp.at[w, pl.ds((g + 2) * NBUF, NBUF)],
                                 idst.at[p], isem.at[p, 1])
        return carry

    lax.fori_loop(0, NITER, it, 0)
    plsc.subcore_barrier()
    pltpu.sync_copy(acc_sh.at[pl.ds(s * RPT, RPT)],
                    out_hbm.at[c, pl.ds(s * RPT, RPT)])


@functools.lru_cache(maxsize=None)
def _spmm_kernel():
    mesh = plsc.VectorSubcoreMesh(core_axis_name="c", subcore_axis_name="s")
    return pl.kernel(
        _spmm_body,
        out_type=jax.ShapeDtypeStruct((NCORES, ACC_N, C), jnp.float32),
        mesh=mesh,
        scratch_types=[
            pltpu.VMEM((2, NBUF, CHUNK), jnp.int32),
            pltpu.VMEM((2, NBUF, CHUNK), jnp.int32),
            pltpu.VMEM((NBUF, CHUNK, C), jnp.float32),
            pltpu.VMEM_SHARED((ACC_N, C), jnp.float32),
            pltpu.SemaphoreType.DMA((2, 2)),
            pltpu.SemaphoreType.DMA((NBUF,)),
            pltpu.SemaphoreType.DMA((NBUF,)),
        ],
    )


def _spmm_call(zs, srcp, dstp, zeros_c):
    return _spmm_kernel()(zs, srcp, dstp, zeros_c)


def _bn_matmul(u, dinv, gamma, beta, w):
    mean = jnp.mean(u, axis=0, keepdims=True)
    var = jnp.mean(u * u, axis=0, keepdims=True) - mean * mean
    hbn = (u - mean) * lax.rsqrt(var + EPS) * gamma + beta
    return dinv * jnp.dot(hbn, w, preferred_element_type=jnp.float32)


def _tc0_body(x_ref, emb_ref, cnt_ref, g_ref, be_ref, w_ref,
              zs_ref, h0_ref, dinv_ref):
    xv = x_ref[...]
    oh = (xv == lax.broadcasted_iota(jnp.int32, (N, NTYPES), 1))
    h0 = jnp.dot(oh.astype(jnp.float32), emb_ref[...],
                 preferred_element_type=jnp.float32)
    deg = cnt_ref[0, :N, 0:1] + cnt_ref[1, :N, 0:1] + 1.0
    dinv = lax.rsqrt(deg)
    zs_ref[...] = _bn_matmul(h0, dinv, g_ref[...], be_ref[...], w_ref[...])
    h0_ref[...] = h0
    dinv_ref[...] = dinv


def _tc0_call(x2, emb, cnt, g, be, w):
    return pl.pallas_call(
        _tc0_body,
        out_shape=[
            jax.ShapeDtypeStruct((N, C), jnp.float32),
            jax.ShapeDtypeStruct((N, C), jnp.float32),
            jax.ShapeDtypeStruct((N, 1), jnp.float32),
        ],
    )(x2, emb, cnt, g, be, w)


def _tc_mid_body(residual, emit_before, s2_ref, zs_ref, dinv_ref, b_ref,
                 g_ref, be_ref, w_ref, *rest):
    if residual:
        before_ref = rest[0]
        rest = rest[1:]
    if emit_before:
        zso_ref, bo_ref = rest
    else:
        (zso_ref,) = rest
    dinv = dinv_ref[...]
    u = (s2_ref[0, :N, :] + s2_ref[1, :N, :] + zs_ref[...]) * dinv + b_ref[...]
    u = jnp.maximum(u, 0.0)
    if residual:
        u = u + before_ref[...]
    if emit_before:
        bo_ref[...] = u
    zso_ref[...] = _bn_matmul(u, dinv, g_ref[...], be_ref[...], w_ref[...])


def _tc_mid_call(s2, zs, dinv, b, g, be, w, before, emit_before):
    residual = before is not None
    outs = [jax.ShapeDtypeStruct((N, C), jnp.float32)]
    if emit_before:
        outs.append(jax.ShapeDtypeStruct((N, C), jnp.float32))
    args = [s2, zs, dinv, b, g, be, w]
    if residual:
        args.append(before)
    return pl.pallas_call(
        functools.partial(_tc_mid_body, residual, emit_before),
        out_shape=outs,
    )(*args)


def _tc_final_body(s2_ref, zs_ref, dinv_ref, b_ref, before_ref, batch_ref,
                   feat_ref, hw_ref, hb_ref, fcw_ref, fcb_ref, out_ref):
    dinv = dinv_ref[...]
    u = (s2_ref[0, :N, :] + s2_ref[1, :N, :] + zs_ref[...]) * dinv + b_ref[...]
    u = jnp.maximum(u, 0.0) + before_ref[...]
    bv = batch_ref[...]
    oh = (bv == lax.broadcasted_iota(jnp.int32, (N, NG), 1)).astype(jnp.float32)
    pooled = lax.dot_general(oh, u, (((0,), (0,)), ((), ())),
                             preferred_element_type=jnp.float32)
    counts = lax.dot_general(oh, jnp.ones((N, 1), jnp.float32),
                             (((0,), (0,)), ((), ())),
                             preferred_element_type=jnp.float32)
    mean = pooled / jnp.maximum(counts, 1.0)
    g = jnp.concatenate([mean, feat_ref[...]], axis=1)
    hid = jnp.maximum(
        jnp.dot(g, hw_ref[...], preferred_element_type=jnp.float32)
        + hb_ref[...], 0.0)
    out_ref[...] = (jnp.dot(hid, fcw_ref[...],
                            preferred_element_type=jnp.float32) + fcb_ref[...])


def _tc_final_call(s2, zs, dinv, b, before, batch2, feats, hw, hb, fcw, fcb):
    return pl.pallas_call(
        _tc_final_body,
        out_shape=jax.ShapeDtypeStruct((NG, C), jnp.float32),
    )(s2, zs, dinv, b, before, batch2, feats, hw, hb, fcw, fcb)


def kernel(x, edge_index, features, batch, emb, bn_gamma, bn_beta,
           conv_w, conv_b, hidden_w, hidden_b, fc_w, fc_b):
    src = edge_index[0].astype(jnp.int32)
    dst = edge_index[1].astype(jnp.int32)
    pad = EPAD - E
    srcp = jnp.concatenate([src, jnp.zeros((pad,), jnp.int32)])
    dstp = jnp.concatenate([dst, jnp.full((pad,), DUMMY, jnp.int32)])
    srcp = srcp.reshape(NWORK, NCHUNK, CHUNK)
    dstp = dstp.reshape(NWORK, NCHUNK, CHUNK)
    zeros_c = jnp.zeros((RPT, C), jnp.float32)
    ones_nc = jnp.ones((N, C), jnp.float32)
    x2 = x.reshape(N, 1).astype(jnp.int32)
    batch2 = batch.reshape(N, 1).astype(jnp.int32)
    gam = bn_gamma.reshape(8, 1, C)
    bet = bn_beta.reshape(8, 1, C)
    ws = conv_w.reshape(8, C, C)
    bs = conv_b.reshape(8, 1, C)
    hb = hidden_b.reshape(1, HID)
    fcw = jnp.pad(fc_w, ((0, 0), (0, C - NT)))
    fcb = jnp.pad(fc_b, (0, C - NT)).reshape(1, C)

    cnt = _spmm_call(ones_nc, srcp, dstp, zeros_c)
    zs, before, dinv = _tc0_call(x2, emb, cnt, gam[0], bet[0], ws[0])
    for k in range(8):
        s2 = _spmm_call(zs, srcp, dstp, zeros_c)
        if k < 7:
            res = (k % 2 == 1)
            emit = ((k + 1) % 2 == 0)
            outs = _tc_mid_call(s2, zs, dinv, bs[k], gam[k + 1], bet[k + 1],
                                ws[k + 1], before if res else None, emit)
            if emit:
                zs, before = outs
            else:
                (zs,) = outs
        else:
            out = _tc_final_call(s2, zs, dinv, bs[k], before, batch2,
                                 features, hidden_w, hb, fcw, fcb)
    return out[:, :NT]


# speedup vs baseline: 1.0001x; 1.0001x over previous
import functools

import jax
import jax.numpy as jnp
from jax import lax
from jax.experimental import pallas as pl
from jax.experimental.pallas import tpu as pltpu
from jax.experimental.pallas import tpu_sc as plsc

N = 10000
E = 320000
C = 128
HID = 256
NT = 10
NTYPES = 32
NF = 16
NG = 64
EPS = 1e-5

NCORES = 2
NSUB = 16
NWORK = NCORES * NSUB
CHUNK = 128
NBUF = 2
EPW = 10240
EPAD = EPW * NWORK
NCHUNK = EPW // CHUNK
NGRP = NCHUNK // NBUF
ACC_N = 10240
RPT = ACC_N // NSUB
DUMMY = N


def _spmm_body(zs_hbm, srcp, dstp, zeros_hbm, out_hbm,
               isrc, idst, rows, acc_sh, isem, gsem, ssem):
    c = lax.axis_index("c")
    s = lax.axis_index("s")
    w = c * NSUB + s
    for p in range(2):
        pltpu.async_copy(srcp.at[w, pl.ds(p * NBUF, NBUF)], isrc.at[p],
                         isem.at[p, 0])
        pltpu.async_copy(dstp.at[w, pl.ds(p * NBUF, NBUF)], idst.at[p],
                         isem.at[p, 1])
    pltpu.sync_copy(zeros_hbm, acc_sh.at[pl.ds(s * RPT, RPT)])
    plsc.subcore_barrier()

    def grp(g, carry):
        p = lax.rem(g, 2)
        for b in range(NBUF):
            @pl.when(g > 0)
            def _():
                pltpu.make_async_copy(rows.at[b], acc_sh.at[idst.at[1 - p, b]],
                                      ssem.at[b]).wait()
        @pl.when(g + 1 < NGRP)
        def _():
            pltpu.async_copy(srcp.at[w, pl.ds((g + 1) * NBUF, NBUF)],
                             isrc.at[1 - p], isem.at[1 - p, 0])
            pltpu.async_copy(dstp.at[w, pl.ds((g + 1) * NBUF, NBUF)],
                             idst.at[1 - p], isem.at[1 - p, 1])
        pltpu.make_async_copy(srcp.at[w, pl.ds(0, NBUF)],
                              isrc.at[p], isem.at[p, 0]).wait()
        pltpu.make_async_copy(dstp.at[w, pl.ds(0, NBUF)],
                              idst.at[p], isem.at[p, 1]).wait()
        for b in range(NBUF):
            pltpu.async_copy(zs_hbm.at[isrc.at[p, b]], rows.at[b], gsem.at[b])
        for b in range(NBUF):
            pltpu.make_async_copy(zs_hbm.at[isrc.at[p, b]], rows.at[b],
                                  gsem.at[b]).wait()
            pltpu.async_copy(rows.at[b], acc_sh.at[idst.at[p, b]],
                             ssem.at[b], add=True)
        return carry

    lax.fori_loop(0, NGRP, grp, 0)
    lastp = (NGRP - 1) % 2
    for b in range(NBUF):
        pltpu.make_async_copy(rows.at[b], acc_sh.at[idst.at[lastp, b]],
                              ssem.at[b]).wait()
    plsc.subcore_barrier()
    pltpu.sync_copy(acc_sh.at[pl.ds(s * RPT, RPT)],
                    out_hbm.at[c, pl.ds(s * RPT, RPT)])


@functools.lru_cache(maxsize=None)
def _spmm_kernel():
    mesh = plsc.VectorSubcoreMesh(core_axis_name="c", subcore_axis_name="s")
    return pl.kernel(
        _spmm_body,
        out_type=jax.ShapeDtypeStruct((NCORES, ACC_N, C), jnp.float32),
        mesh=mesh,
        scratch_types=[
            pltpu.VMEM((2, NBUF, CHUNK), jnp.int32),
            pltpu.VMEM((2, NBUF, CHUNK), jnp.int32),
            pltpu.VMEM((NBUF, CHUNK, C), jnp.float32),
            pltpu.VMEM_SHARED((ACC_N, C), jnp.float32),
            pltpu.SemaphoreType.DMA((2, 2)),
            pltpu.SemaphoreType.DMA((NBUF,)),
            pltpu.SemaphoreType.DMA((NBUF,)),
        ],
    )


def _spmm_call(zs, srcp, dstp, zeros_c):
    return _spmm_kernel()(zs, srcp, dstp, zeros_c)


def _bn_matmul(u, dinv, gamma, beta, w):
    mean = jnp.mean(u, axis=0, keepdims=True)
    var = jnp.mean(u * u, axis=0, keepdims=True) - mean * mean
    hbn = (u - mean) * lax.rsqrt(var + EPS) * gamma + beta
    return dinv * jnp.dot(hbn, w, preferred_element_type=jnp.float32)


def _tc0_body(x_ref, emb_ref, cnt_ref, g_ref, be_ref, w_ref,
              zs_ref, h0_ref, dinv_ref):
    xv = x_ref[...]
    oh = (xv == lax.broadcasted_iota(jnp.int32, (N, NTYPES), 1))
    h0 = jnp.dot(oh.astype(jnp.float32), emb_ref[...],
                 preferred_element_type=jnp.float32)
    deg = cnt_ref[0, :N, 0:1] + cnt_ref[1, :N, 0:1] + 1.0
    dinv = lax.rsqrt(deg)
    zs_ref[...] = _bn_matmul(h0, dinv, g_ref[...], be_ref[...], w_ref[...])
    h0_ref[...] = h0
    dinv_ref[...] = dinv


def _tc0_call(x2, emb, cnt, g, be, w):
    return pl.pallas_call(
        _tc0_body,
        out_shape=[
            jax.ShapeDtypeStruct((N, C), jnp.float32),
            jax.ShapeDtypeStruct((N, C), jnp.float32),
            jax.ShapeDtypeStruct((N, 1), jnp.float32),
        ],
    )(x2, emb, cnt, g, be, w)


def _tc_mid_body(residual, emit_before, s2_ref, zs_ref, dinv_ref, b_ref,
                 g_ref, be_ref, w_ref, *rest):
    if residual:
        before_ref = rest[0]
        rest = rest[1:]
    if emit_before:
        zso_ref, bo_ref = rest
    else:
        (zso_ref,) = rest
    dinv = dinv_ref[...]
    u = (s2_ref[0, :N, :] + s2_ref[1, :N, :] + zs_ref[...]) * dinv + b_ref[...]
    u = jnp.maximum(u, 0.0)
    if residual:
        u = u + before_ref[...]
    if emit_before:
        bo_ref[...] = u
    zso_ref[...] = _bn_matmul(u, dinv, g_ref[...], be_ref[...], w_ref[...])


def _tc_mid_call(s2, zs, dinv, b, g, be, w, before, emit_before):
    residual = before is not None
    outs = [jax.ShapeDtypeStruct((N, C), jnp.float32)]
    if emit_before:
        outs.append(jax.ShapeDtypeStruct((N, C), jnp.float32))
    args = [s2, zs, dinv, b, g, be, w]
    if residual:
        args.append(before)
    return pl.pallas_call(
        functools.partial(_tc_mid_body, residual, emit_before),
        out_shape=outs,
    )(*args)


def _tc_final_body(s2_ref, zs_ref, dinv_ref, b_ref, before_ref, batch_ref,
                   feat_ref, hw_ref, hb_ref, fcw_ref, fcb_ref, out_ref):
    dinv = dinv_ref[...]
    u = (s2_ref[0, :N, :] + s2_ref[1, :N, :] + zs_ref[...]) * dinv + b_ref[...]
    u = jnp.maximum(u, 0.0) + before_ref[...]
    bv = batch_ref[...]
    oh = (bv == lax.broadcasted_iota(jnp.int32, (N, NG), 1)).astype(jnp.float32)
    pooled = lax.dot_general(oh, u, (((0,), (0,)), ((), ())),
                             preferred_element_type=jnp.float32)
    counts = lax.dot_general(oh, jnp.ones((N, 1), jnp.float32),
                             (((0,), (0,)), ((), ())),
                             preferred_element_type=jnp.float32)
    mean = pooled / jnp.maximum(counts, 1.0)
    g = jnp.concatenate([mean, feat_ref[...]], axis=1)
    hid = jnp.maximum(
        jnp.dot(g, hw_ref[...], preferred_element_type=jnp.float32)
        + hb_ref[...], 0.0)
    out_ref[...] = (jnp.dot(hid, fcw_ref[...],
                            preferred_element_type=jnp.float32) + fcb_ref[...])


def _tc_final_call(s2, zs, dinv, b, before, batch2, feats, hw, hb, fcw, fcb):
    return pl.pallas_call(
        _tc_final_body,
        out_shape=jax.ShapeDtypeStruct((NG, C), jnp.float32),
    )(s2, zs, dinv, b, before, batch2, feats, hw, hb, fcw, fcb)


def kernel(x, edge_index, features, batch, emb, bn_gamma, bn_beta,
           conv_w, conv_b, hidden_w, hidden_b, fc_w, fc_b):
    src = edge_index[0].astype(jnp.int32)
    dst = edge_index[1].astype(jnp.int32)
    pad = EPAD - E
    srcp = jnp.concatenate([src, jnp.zeros((pad,), jnp.int32)])
    dstp = jnp.concatenate([dst, jnp.full((pad,), DUMMY, jnp.int32)])
    srcp = srcp.reshape(NWORK, NCHUNK, CHUNK)
    dstp = dstp.reshape(NWORK, NCHUNK, CHUNK)
    zeros_c = jnp.zeros((RPT, C), jnp.float32)
    ones_nc = jnp.ones((N, C), jnp.float32)
    x2 = x.reshape(N, 1).astype(jnp.int32)
    batch2 = batch.reshape(N, 1).astype(jnp.int32)
    gam = bn_gamma.reshape(8, 1, C)
    bet = bn_beta.reshape(8, 1, C)
    ws = conv_w.reshape(8, C, C)
    bs = conv_b.reshape(8, 1, C)
    hb = hidden_b.reshape(1, HID)
    fcw = jnp.pad(fc_w, ((0, 0), (0, C - NT)))
    fcb = jnp.pad(fc_b, (0, C - NT)).reshape(1, C)

    cnt = _spmm_call(ones_nc, srcp, dstp, zeros_c)
    zs, before, dinv = _tc0_call(x2, emb, cnt, gam[0], bet[0], ws[0])
    for k in range(8):
        s2 = _spmm_call(zs, srcp, dstp, zeros_c)
        if k < 7:
            res = (k % 2 == 1)
            emit = ((k + 1) % 2 == 0)
            outs = _tc_mid_call(s2, zs, dinv, bs[k], gam[k + 1], bet[k + 1],
                                ws[k + 1], before if res else None, emit)
            if emit:
                zs, before = outs
            else:
                (zs,) = outs
        else:
            out = _tc_final_call(s2, zs, dinv, bs[k], before, batch2,
                                 features, hidden_w, hb, fcw, fcb)
    return out[:, :NT]


# speedup vs baseline: 1.0667x; 1.0665x over previous
import functools

import jax
import jax.numpy as jnp
from jax import lax
from jax.experimental import pallas as pl
from jax.experimental.pallas import tpu as pltpu
from jax.experimental.pallas import tpu_sc as plsc

N = 10000
E = 320000
C = 128
HID = 256
NT = 10
NTYPES = 32
NF = 16
NG = 64
EPS = 1e-5

NCORES = 2
NSUB = 16
NWORK = NCORES * NSUB
CHUNK = 128
EPW = 10240
EPAD = EPW * NWORK
NCHUNK = EPW // CHUNK
ACC_N = 10240
RPT = ACC_N // NSUB
DUMMY = N


def _spmm_body(zs_hbm, srcp, dstp, zeros_hbm, out_hbm,
               src0, src1, dst0, dst1, rows, acc_sh, gsem):
    c = lax.axis_index("c")
    s = lax.axis_index("s")
    w = c * NSUB + s
    pltpu.sync_copy(zeros_hbm, acc_sh.at[pl.ds(s * RPT, RPT)])
    plsc.subcore_barrier()
    base = w * EPW
    srcv = (src0, src1)
    dstv = (dst0, dst1)

    pltpu.sync_copy(srcp.at[pl.ds(base, CHUNK)], src0)
    pltpu.sync_copy(dstp.at[pl.ds(base, CHUNK)], dst0)
    pltpu.async_copy(zs_hbm.at[src0], rows.at[0], gsem.at[0])

    def pair(k, carry):
        for b in range(2):
            i = 2 * k + b
            nb = 1 - b

            @pl.when(i + 1 < NCHUNK)
            def _():
                pltpu.sync_copy(srcp.at[pl.ds(base + (i + 1) * CHUNK, CHUNK)],
                                srcv[nb])
                pltpu.sync_copy(dstp.at[pl.ds(base + (i + 1) * CHUNK, CHUNK)],
                                dstv[nb])
            pltpu.make_async_copy(zs_hbm.at[srcv[b]], rows.at[b],
                                  gsem.at[b]).wait()

            @pl.when(i + 1 < NCHUNK)
            def _():
                pltpu.async_copy(zs_hbm.at[srcv[nb]], rows.at[nb],
                                 gsem.at[nb])
            pltpu.sync_copy(rows.at[b], acc_sh.at[dstv[b]], add=True)
        return carry

    lax.fori_loop(0, NCHUNK // 2, pair, 0)
    plsc.subcore_barrier()
    pltpu.sync_copy(acc_sh.at[pl.ds(s * RPT, RPT)],
                    out_hbm.at[c, pl.ds(s * RPT, RPT)])


@functools.lru_cache(maxsize=None)
def _spmm_kernel():
    mesh = plsc.VectorSubcoreMesh(core_axis_name="c", subcore_axis_name="s")
    return pl.kernel(
        _spmm_body,
        out_type=jax.ShapeDtypeStruct((NCORES, ACC_N, C), jnp.float32),
        mesh=mesh,
        scratch_types=[
            pltpu.VMEM((CHUNK,), jnp.int32),
            pltpu.VMEM((CHUNK,), jnp.int32),
            pltpu.VMEM((CHUNK,), jnp.int32),
            pltpu.VMEM((CHUNK,), jnp.int32),
            pltpu.VMEM((2, CHUNK, C), jnp.float32),
            pltpu.VMEM_SHARED((ACC_N, C), jnp.float32),
            pltpu.SemaphoreType.DMA((2,)),
        ],
    )


def _spmm_call(zs, srcp, dstp, zeros_c):
    return _spmm_kernel()(zs, srcp, dstp, zeros_c)


def _bn_matmul(u, dinv, gamma, beta, w):
    mean = jnp.mean(u, axis=0, keepdims=True)
    var = jnp.mean(u * u, axis=0, keepdims=True) - mean * mean
    hbn = (u - mean) * lax.rsqrt(var + EPS) * gamma + beta
    return dinv * jnp.dot(hbn, w, preferred_element_type=jnp.float32)


def _tc0_body(x_ref, emb_ref, cnt_ref, g_ref, be_ref, w_ref,
              zs_ref, h0_ref, dinv_ref):
    xv = x_ref[...]
    oh = (xv == lax.broadcasted_iota(jnp.int32, (N, NTYPES), 1))
    h0 = jnp.dot(oh.astype(jnp.float32), emb_ref[...],
                 preferred_element_type=jnp.float32)
    deg = cnt_ref[0, :N, 0:1] + cnt_ref[1, :N, 0:1] + 1.0
    dinv = lax.rsqrt(deg)
    zs_ref[...] = _bn_matmul(h0, dinv, g_ref[...], be_ref[...], w_ref[...])
    h0_ref[...] = h0
    dinv_ref[...] = dinv


def _tc0_call(x2, emb, cnt, g, be, w):
    return pl.pallas_call(
        _tc0_body,
        out_shape=[
            jax.ShapeDtypeStruct((N, C), jnp.float32),
            jax.ShapeDtypeStruct((N, C), jnp.float32),
            jax.ShapeDtypeStruct((N, 1), jnp.float32),
        ],
    )(x2, emb, cnt, g, be, w)


def _tc_mid_body(residual, emit_before, s2_ref, zs_ref, dinv_ref, b_ref,
                 g_ref, be_ref, w_ref, *rest):
    if residual:
        before_ref = rest[0]
        rest = rest[1:]
    if emit_before:
        zso_ref, bo_ref = rest
    else:
        (zso_ref,) = rest
    dinv = dinv_ref[...]
    u = (s2_ref[0, :N, :] + s2_ref[1, :N, :] + zs_ref[...]) * dinv + b_ref[...]
    u = jnp.maximum(u, 0.0)
    if residual:
        u = u + before_ref[...]
    if emit_before:
        bo_ref[...] = u
    zso_ref[...] = _bn_matmul(u, dinv, g_ref[...], be_ref[...], w_ref[...])


def _tc_mid_call(s2, zs, dinv, b, g, be, w, before, emit_before):
    residual = before is not None
    outs = [jax.ShapeDtypeStruct((N, C), jnp.float32)]
    if emit_before:
        outs.append(jax.ShapeDtypeStruct((N, C), jnp.float32))
    args = [s2, zs, dinv, b, g, be, w]
    if residual:
        args.append(before)
    return pl.pallas_call(
        functools.partial(_tc_mid_body, residual, emit_before),
        out_shape=outs,
    )(*args)


def _tc_final_body(s2_ref, zs_ref, dinv_ref, b_ref, before_ref, batch_ref,
                   feat_ref, hw_ref, hb_ref, fcw_ref, fcb_ref, out_ref):
    dinv = dinv_ref[...]
    u = (s2_ref[0, :N, :] + s2_ref[1, :N, :] + zs_ref[...]) * dinv + b_ref[...]
    u = jnp.maximum(u, 0.0) + before_ref[...]
    bv = batch_ref[...]
    oh = (bv == lax.broadcasted_iota(jnp.int32, (N, NG), 1)).astype(jnp.float32)
    pooled = lax.dot_general(oh, u, (((0,), (0,)), ((), ())),
                             preferred_element_type=jnp.float32)
    counts = lax.dot_general(oh, jnp.ones((N, 1), jnp.float32),
                             (((0,), (0,)), ((), ())),
                             preferred_element_type=jnp.float32)
    mean = pooled / jnp.maximum(counts, 1.0)
    g = jnp.concatenate([mean, feat_ref[...]], axis=1)
    hid = jnp.maximum(
        jnp.dot(g, hw_ref[...], preferred_element_type=jnp.float32)
        + hb_ref[...], 0.0)
    out_ref[...] = (jnp.dot(hid, fcw_ref[...],
                            preferred_element_type=jnp.float32) + fcb_ref[...])


def _tc_final_call(s2, zs, dinv, b, before, batch2, feats, hw, hb, fcw, fcb):
    return pl.pallas_call(
        _tc_final_body,
        out_shape=jax.ShapeDtypeStruct((NG, C), jnp.float32),
    )(s2, zs, dinv, b, before, batch2, feats, hw, hb, fcw, fcb)


def kernel(x, edge_index, features, batch, emb, bn_gamma, bn_beta,
           conv_w, conv_b, hidden_w, hidden_b, fc_w, fc_b):
    src = edge_index[0].astype(jnp.int32)
    dst = edge_index[1].astype(jnp.int32)
    pad = EPAD - E
    srcp = jnp.concatenate([src, jnp.zeros((pad,), jnp.int32)])
    dstp = jnp.concatenate([dst, jnp.full((pad,), DUMMY, jnp.int32)])
    zeros_c = jnp.zeros((RPT, C), jnp.float32)
    ones_nc = jnp.ones((N, C), jnp.float32)
    x2 = x.reshape(N, 1).astype(jnp.int32)
    batch2 = batch.reshape(N, 1).astype(jnp.int32)
    gam = bn_gamma.reshape(8, 1, C)
    bet = bn_beta.reshape(8, 1, C)
    ws = conv_w.reshape(8, C, C)
    bs = conv_b.reshape(8, 1, C)
    hb = hidden_b.reshape(1, HID)
    fcw = jnp.pad(fc_w, ((0, 0), (0, C - NT)))
    fcb = jnp.pad(fc_b, (0, C - NT)).reshape(1, C)

    cnt = _spmm_call(ones_nc, srcp, dstp, zeros_c)
    zs, before, dinv = _tc0_call(x2, emb, cnt, gam[0], bet[0], ws[0])
    for k in range(8):
        s2 = _spmm_call(zs, srcp, dstp, zeros_c)
        if k < 7:
            res = (k % 2 == 1)
            emit = ((k + 1) % 2 == 0)
            outs = _tc_mid_call(s2, zs, dinv, bs[k], gam[k + 1], bet[k + 1],
                                ws[k + 1], before if res else None, emit)
            if emit:
                zs, before = outs
            else:
                (zs,) = outs
        else:
            out = _tc_final_call(s2, zs, dinv, bs[k], before, batch2,
                                 features, hidden_w, hb, fcw, fcb)
    return out[:, :NT]
